# baseline (speedup 1.0000x reference)
import jax
import jax.numpy as jnp
from jax import lax
from jax.experimental import pallas as pl
from jax.experimental.pallas import tpu as pltpu
from jax.experimental.pallas import tpu_sc as plsc

VOCAB_SIZE = 100000
EMBED_DIM = 64
MAX_LEN = 200
BATCH = 4096
SEQ_LEN = 200

NUM_WORKERS = 32
SEQ_PER_W = BATCH // NUM_WORKERS
HALF = SEQ_LEN // 2
NB = 4


def _emb_body(x_hbm, tok_hbm, pos_hbm, out_hbm, pos_sh, idx_v,
              b0, b1, b2, b3,
              g0, g1, g2, g3, s0, s1, s2, s3, p0, p1, p2, p3):
    c = lax.axis_index("c")
    s = lax.axis_index("s")
    wid = s * 2 + c
    row0 = wid * SEQ_PER_W * SEQ_LEN

    bufs = (b0, b1, b2, b3)
    gsems = (g0, g1, g2, g3)
    ssems = (s0, s1, s2, s3)
    psems = (p0, p1, p2, p3)

    @pl.when(s == 0)
    def _():
        pltpu.sync_copy(pos_hbm, pos_sh)

    plsc.subcore_barrier()

    def prefill(p):
        pltpu.async_copy(pos_sh, bufs[p], psems[p])

    def wait_prefill(p):
        pltpu.make_async_copy(pos_sh, bufs[p], psems[p]).wait()

    def gather_add(g, p):
        pltpu.async_copy(tok_hbm.at[idx_v.at[g, 0]],
                         bufs[p].at[pl.ds(0, HALF)], gsems[p], add=True)
        pltpu.async_copy(tok_hbm.at[idx_v.at[g, 1]],
                         bufs[p].at[pl.ds(HALF, HALF)], gsems[p], add=True)

    def wait_gather(g, p):
        pltpu.make_async_copy(tok_hbm.at[idx_v.at[g, 0]],
                              bufs[p].at[pl.ds(0, HALF)], gsems[p]).wait()
        pltpu.make_async_copy(tok_hbm.at[idx_v.at[g, 1]],
                              bufs[p].at[pl.ds(HALF, HALF)], gsems[p]).wait()

    def store(g, p):
        pltpu.async_copy(
            bufs[p], out_hbm.at[pl.ds(row0 + g * SEQ_LEN, SEQ_LEN)], ssems[p])

    def wait_store(g, p):
        pltpu.make_async_copy(
            bufs[p], out_hbm.at[pl.ds(row0 + g * SEQ_LEN, SEQ_LEN)],
            ssems[p]).wait()

    pltpu.sync_copy(x_hbm.at[pl.ds(wid * SEQ_PER_W, SEQ_PER_W)], idx_v)
    prefill(0)
    prefill(1)
    prefill(2)
    wait_prefill(0)
    gather_add(0, 0)
    wait_prefill(1)
    gather_add(1, 1)

    def slot(g, p):
        q = (p + 3) % NB
        @pl.when(g >= 1)
        def _():
            wait_store(g - 1, q)
        @pl.when(g + 3 < SEQ_PER_W)
        def _():
            prefill(q)
        @pl.when(g + 2 < SEQ_PER_W)
        def _():
            r = (p + 2) % NB
            wait_prefill(r)
            gather_add(g + 2, r)
        wait_gather(g, p)
        store(g, p)

    def step(t, carry):
        for k in range(NB):
            slot(NB * t + k, k)
        return carry

    lax.fori_loop(0, SEQ_PER_W // NB, step, 0)

    wait_store(SEQ_PER_W - 1, (SEQ_PER_W - 1) % NB)


@jax.jit
def kernel(x, token_table, pos_table):
    x3 = x.astype(jnp.int32).reshape(BATCH, 2, HALF)
    mesh = plsc.VectorSubcoreMesh(core_axis_name="c", subcore_axis_name="s")
    out_flat = pl.kernel(
        _emb_body,
        out_type=jax.ShapeDtypeStruct((BATCH * SEQ_LEN, EMBED_DIM),
                                      jnp.float32),
        mesh=mesh,
        scratch_types=[
            pltpu.VMEM_SHARED((MAX_LEN, EMBED_DIM), jnp.float32),
            pltpu.VMEM((SEQ_PER_W, 2, HALF), jnp.int32),
            pltpu.VMEM((SEQ_LEN, EMBED_DIM), jnp.float32),
            pltpu.VMEM((SEQ_LEN, EMBED_DIM), jnp.float32),
            pltpu.VMEM((SEQ_LEN, EMBED_DIM), jnp.float32),
            pltpu.VMEM((SEQ_LEN, EMBED_DIM), jnp.float32),
        ] + [pltpu.SemaphoreType.DMA] * 12,
        compiler_params=pltpu.CompilerParams(use_tc_tiling_on_sc=False),
    )(x3, token_table, pos_table)
    return out_flat.reshape(BATCH, SEQ_LEN, EMBED_DIM)
